# baseline (device time: 30064 ns/iter reference)
import jax
import jax.numpy as jnp
from jax import lax
from jax.experimental import pallas as pl
from jax.experimental.pallas import tpu as pltpu

N_DEV = 4
B, SQ, SKV, HQ, DH = 2, 128, 512, 4, 64
D_MODEL = 512
D_QK = HQ * DH
CH = SKV // N_DEV
NBH = B * HQ


def kernel(x, Wq, K_ext, V_ext, Wo):
    def body(x_ref, wq_ref, k_ref, v_ref, wo_ref, out_ref,
             ctx_send, ctx_recv, st_send, st_recv,
             cs_sems, cr_sems, ss_sems, sr_sems):
        my_pos = lax.axis_index("i")
        p_step = [jnp.bitwise_xor(my_pos, 1), 3 - my_pos]

        barrier_sem = pltpu.get_barrier_semaphore()
        for nbr in p_step:
            pl.semaphore_signal(barrier_sem, inc=1, device_id=(nbr,),
                                device_id_type=pl.DeviceIdType.MESH)
        pl.semaphore_wait(barrier_sem, 2)

        x_flat = x_ref[...].reshape(B * SQ, D_MODEL).astype(jnp.bfloat16)
        q_all = jnp.dot(x_flat, wq_ref[...].astype(jnp.bfloat16),
                        preferred_element_type=jnp.float32)

        k_loc = k_ref[...].reshape(B * CH, D_QK)
        v_loc = v_ref[...].reshape(B * CH, D_QK)

        qb = lax.broadcasted_iota(jnp.int32, (SQ, CH), 0) // 64
        kbg = my_pos * 2 + lax.broadcasted_iota(jnp.int32, (SQ, CH), 1) // 64
        mask = (qb == kbg) | (kbg == 0) | ((qb + kbg) % 3 == 0)

        ctx_blocks, m_blocks, l_blocks = [], [], []
        for b in range(B):
            for hh in range(HQ):
                q = q_all[b * SQ:(b + 1) * SQ, hh * DH:(hh + 1) * DH]
                kmat = k_loc[b * CH:(b + 1) * CH,
                             hh * DH:(hh + 1) * DH].astype(jnp.float32)
                vmat = v_loc[b * CH:(b + 1) * CH,
                             hh * DH:(hh + 1) * DH].astype(jnp.float32)
                s = lax.dot_general(
                    q, kmat, (((1,), (1,)), ((), ())),
                    preferred_element_type=jnp.float32) * 0.125
                s = jnp.where(mask, s, -1e9)
                m = jnp.max(s, axis=1, keepdims=True)
                w = jnp.exp(s - m)
                l = jnp.sum(w, axis=1, keepdims=True)
                ctx = jnp.dot(w, vmat, preferred_element_type=jnp.float32)
                ctx_blocks.append(ctx[None])
                m_blocks.append(m[None])
                l_blocks.append(l[None])
        ctx_acc = jnp.concatenate(ctx_blocks, axis=0)
        m_acc = jnp.concatenate(m_blocks, axis=0)
        l_acc = jnp.concatenate(l_blocks, axis=0)

        for s_i in range(2):
            ctx_send[s_i] = ctx_acc.astype(jnp.bfloat16)
            st_send[s_i, :, :, 0:1] = m_acc
            st_send[s_i, :, :, 1:2] = l_acc

            ctx_rdma = pltpu.make_async_remote_copy(
                src_ref=ctx_send.at[s_i], dst_ref=ctx_recv.at[s_i],
                send_sem=cs_sems.at[s_i], recv_sem=cr_sems.at[s_i],
                device_id=(p_step[s_i],),
                device_id_type=pl.DeviceIdType.MESH,
            )
            st_rdma = pltpu.make_async_remote_copy(
                src_ref=st_send.at[s_i], dst_ref=st_recv.at[s_i],
                send_sem=ss_sems.at[s_i], recv_sem=sr_sems.at[s_i],
                device_id=(p_step[s_i],),
                device_id_type=pl.DeviceIdType.MESH,
            )
            ctx_rdma.start()
            st_rdma.start()
            ctx_rdma.wait()
            st_rdma.wait()

            ctx_o = ctx_recv[s_i].astype(jnp.float32)
            m_o = st_recv[s_i, :, :, 0:1]
            l_o = st_recv[s_i, :, :, 1:2]

            m_new = jnp.maximum(m_acc, m_o)
            alpha = jnp.exp(m_acc - m_new)
            beta = jnp.exp(m_o - m_new)
            ctx_acc = alpha * ctx_acc + beta * ctx_o
            l_acc = alpha * l_acc + beta * l_o
            m_acc = m_new

        ctx_n = (ctx_acc / l_acc).astype(jnp.bfloat16)
        wo = wo_ref[...].astype(jnp.bfloat16)
        for b in range(B):
            ctx_b = jnp.concatenate(
                [ctx_n[b * HQ + hh] for hh in range(HQ)], axis=1)
            out_ref[b, :, :] = jnp.dot(ctx_b, wo,
                                       preferred_element_type=jnp.float32)

    return pl.pallas_call(
        body,
        out_shape=jax.ShapeDtypeStruct((B, SQ, D_MODEL), jnp.float32),
        in_specs=[pl.BlockSpec(memory_space=pltpu.VMEM)] * 5,
        out_specs=pl.BlockSpec(memory_space=pltpu.VMEM),
        scratch_shapes=[
            pltpu.VMEM((2, NBH, SQ, DH), jnp.bfloat16),
            pltpu.VMEM((2, NBH, SQ, DH), jnp.bfloat16),
            pltpu.VMEM((2, NBH, SQ, 2), jnp.float32),
            pltpu.VMEM((2, NBH, SQ, 2), jnp.float32),
            pltpu.SemaphoreType.DMA((2,)),
            pltpu.SemaphoreType.DMA((2,)),
            pltpu.SemaphoreType.DMA((2,)),
            pltpu.SemaphoreType.DMA((2,)),
        ],
        compiler_params=pltpu.CompilerParams(collective_id=0),
    )(x, Wq, K_ext, V_ext, Wo)


# device time: 24618 ns/iter; 1.2212x vs baseline; 1.2212x over previous
import jax
import jax.numpy as jnp
from jax import lax
from jax.experimental import pallas as pl
from jax.experimental.pallas import tpu as pltpu

N_DEV = 4
B, SQ, SKV, HQ, DH = 2, 128, 512, 4, 64
D_MODEL = 512
D_QK = HQ * DH
CH = SKV // N_DEV
ROWS = B * CH


def kernel(x, Wq, K_ext, V_ext, Wo):
    def body(x_ref, wq_ref, k_ref, v_ref, wo_ref, out_ref,
             kv_comm, k_all, v_all, send_sems, recv_sems):
        my_pos = lax.axis_index("i")
        left = lax.rem(my_pos + N_DEV - 1, N_DEV)
        right = lax.rem(my_pos + 1, N_DEV)

        barrier_sem = pltpu.get_barrier_semaphore()
        for nbr in (left, right):
            pl.semaphore_signal(barrier_sem, inc=1, device_id=(nbr,),
                                device_id_type=pl.DeviceIdType.MESH)
        pl.semaphore_wait(barrier_sem, 2)

        k_loc = k_ref[...].astype(jnp.bfloat16).reshape(ROWS, D_QK)
        v_loc = v_ref[...].astype(jnp.bfloat16).reshape(ROWS, D_QK)
        kv_comm[0, :ROWS, :] = k_loc
        kv_comm[0, ROWS:, :] = v_loc
        k_all[my_pos, :, :] = k_loc
        v_all[my_pos, :, :] = v_loc

        for h in range(N_DEV - 1):
            s_slot, r_slot = h % 2, (h + 1) % 2
            rdma = pltpu.make_async_remote_copy(
                src_ref=kv_comm.at[s_slot],
                dst_ref=kv_comm.at[r_slot],
                send_sem=send_sems.at[s_slot],
                recv_sem=recv_sems.at[r_slot],
                device_id=(right,),
                device_id_type=pl.DeviceIdType.MESH,
            )
            rdma.start()
            rdma.wait()
            origin = lax.rem(my_pos - h - 1 + N_DEV, N_DEV)
            k_all[origin, :, :] = kv_comm[r_slot, :ROWS, :]
            v_all[origin, :, :] = kv_comm[r_slot, ROWS:, :]

        x_flat = x_ref[...].reshape(B * SQ, D_MODEL)
        q_all = jnp.dot(x_flat, wq_ref[...],
                        preferred_element_type=jnp.float32)

        qi = lax.broadcasted_iota(jnp.int32, (SQ, SKV), 0)
        ki = lax.broadcasted_iota(jnp.int32, (SQ, SKV), 1)
        qb, kb = qi // 64, ki // 64
        mask = (qb == kb) | (kb == 0) | ((qb + kb) % 3 == 0)

        wo = wo_ref[...]
        for b in range(B):
            acc = jnp.zeros((SQ, D_MODEL), jnp.float32)
            for hh in range(HQ):
                q = q_all[b * SQ:(b + 1) * SQ, hh * DH:(hh + 1) * DH]
                kmat = jnp.concatenate(
                    [k_all[o, b * CH:(b + 1) * CH, hh * DH:(hh + 1) * DH]
                     for o in range(N_DEV)], axis=0).astype(jnp.float32)
                vmat = jnp.concatenate(
                    [v_all[o, b * CH:(b + 1) * CH, hh * DH:(hh + 1) * DH]
                     for o in range(N_DEV)], axis=0).astype(jnp.float32)
                s = lax.dot_general(
                    q, kmat, (((1,), (1,)), ((), ())),
                    preferred_element_type=jnp.float32) * 0.125
                s = jnp.where(mask, s, -1e9)
                m = jnp.max(s, axis=1, keepdims=True)
                w = jnp.exp(s - m)
                w = w / jnp.sum(w, axis=1, keepdims=True)
                ctx = jnp.dot(w, vmat,
                              preferred_element_type=jnp.float32)
                acc = acc + jnp.dot(
                    ctx, wo[hh * DH:(hh + 1) * DH, :],
                    preferred_element_type=jnp.float32)
            out_ref[b, :, :] = acc

    return pl.pallas_call(
        body,
        out_shape=jax.ShapeDtypeStruct((B, SQ, D_MODEL), jnp.float32),
        in_specs=[pl.BlockSpec(memory_space=pltpu.VMEM)] * 5,
        out_specs=pl.BlockSpec(memory_space=pltpu.VMEM),
        scratch_shapes=[
            pltpu.VMEM((2, 2 * ROWS, D_QK), jnp.bfloat16),
            pltpu.VMEM((N_DEV, ROWS, D_QK), jnp.bfloat16),
            pltpu.VMEM((N_DEV, ROWS, D_QK), jnp.bfloat16),
            pltpu.SemaphoreType.DMA((2,)),
            pltpu.SemaphoreType.DMA((2,)),
        ],
        compiler_params=pltpu.CompilerParams(collective_id=0),
    )(x, Wq, K_ext, V_ext, Wo)


# device time: 15202 ns/iter; 1.9776x vs baseline; 1.6194x over previous
import jax
import jax.numpy as jnp
from jax import lax
from jax.experimental import pallas as pl
from jax.experimental.pallas import tpu as pltpu

N_DEV = 4
B, SQ, SKV, HQ, DH = 2, 128, 512, 4, 64
D_MODEL = 512
D_QK = HQ * DH
CH = SKV // N_DEV
NBH = B * HQ


def kernel(x, Wq, K_ext, V_ext, Wo):
    def body(x_hbm, wq_hbm, k_hbm, v_hbm, wo_hbm, out_ref,
             x_vm, wq_vm, k_vm, v_vm, wo_vm,
             ctx_send, ctx_recv, st_send, st_recv,
             in_sems, cs_sems, cr_sems, ss_sems, sr_sems):
        my_pos = lax.axis_index("i")
        p_step = [jnp.bitwise_xor(my_pos, 1), 3 - my_pos]

        in_copies = [
            pltpu.make_async_copy(src, dst, in_sems.at[i])
            for i, (src, dst) in enumerate([
                (x_hbm, x_vm), (wq_hbm, wq_vm), (k_hbm, k_vm),
                (v_hbm, v_vm), (wo_hbm, wo_vm)])
        ]
        for c in in_copies:
            c.start()

        barrier_sem = pltpu.get_barrier_semaphore()
        for nbr in p_step:
            pl.semaphore_signal(barrier_sem, inc=1, device_id=(nbr,),
                                device_id_type=pl.DeviceIdType.MESH)
        pl.semaphore_wait(barrier_sem, 2)

        in_copies[0].wait()
        in_copies[1].wait()
        x_flat = x_vm[...].reshape(B * SQ, D_MODEL).astype(jnp.bfloat16)
        q_all = jnp.dot(x_flat, wq_vm[...].astype(jnp.bfloat16),
                        preferred_element_type=jnp.float32)

        in_copies[2].wait()
        in_copies[3].wait()
        k_loc = k_vm[...].astype(jnp.bfloat16).reshape(B * CH, D_QK)
        v_loc = v_vm[...].astype(jnp.bfloat16).reshape(B * CH, D_QK)

        kbg = my_pos * 2 + lax.broadcasted_iota(jnp.int32, (CH, SQ), 0) // 64
        qb = lax.broadcasted_iota(jnp.int32, (CH, SQ), 1) // 64
        mask = (qb == kbg) | (kbg == 0) | ((qb + kbg) % 3 == 0)

        ctx_blocks, m_blocks, l_blocks = [], [], []
        for b in range(B):
            for hh in range(HQ):
                q = q_all[b * SQ:(b + 1) * SQ,
                          hh * DH:(hh + 1) * DH].astype(jnp.bfloat16)
                kmat = k_loc[b * CH:(b + 1) * CH, hh * DH:(hh + 1) * DH]
                vmat = v_loc[b * CH:(b + 1) * CH, hh * DH:(hh + 1) * DH]
                s = lax.dot_general(
                    kmat, q, (((1,), (1,)), ((), ())),
                    preferred_element_type=jnp.float32) * 0.125
                s = jnp.where(mask, s, -1e9)
                m = jnp.max(s, axis=0, keepdims=True)
                w = jnp.exp(s - m)
                l = jnp.sum(w, axis=0, keepdims=True)
                ctx = lax.dot_general(
                    vmat, w.astype(jnp.bfloat16), (((0,), (0,)), ((), ())),
                    preferred_element_type=jnp.float32)
                ctx_blocks.append(ctx[None])
                m_blocks.append(m[None])
                l_blocks.append(l[None])
        ctx_acc = jnp.concatenate(ctx_blocks, axis=0)
        m_acc = jnp.concatenate(m_blocks, axis=0)
        l_acc = jnp.concatenate(l_blocks, axis=0)

        for s_i in range(2):
            ctx_send[s_i] = ctx_acc.astype(jnp.bfloat16)
            st_send[s_i] = jnp.concatenate([m_acc, l_acc], axis=1)

            ctx_rdma = pltpu.make_async_remote_copy(
                src_ref=ctx_send.at[s_i], dst_ref=ctx_recv.at[s_i],
                send_sem=cs_sems.at[s_i], recv_sem=cr_sems.at[s_i],
                device_id=(p_step[s_i],),
                device_id_type=pl.DeviceIdType.MESH,
            )
            st_rdma = pltpu.make_async_remote_copy(
                src_ref=st_send.at[s_i], dst_ref=st_recv.at[s_i],
                send_sem=ss_sems.at[s_i], recv_sem=sr_sems.at[s_i],
                device_id=(p_step[s_i],),
                device_id_type=pl.DeviceIdType.MESH,
            )
            ctx_rdma.start()
            st_rdma.start()
            ctx_rdma.wait()
            st_rdma.wait()

            ctx_o = ctx_recv[s_i].astype(jnp.float32)
            m_o = st_recv[s_i, :, 0:1, :]
            l_o = st_recv[s_i, :, 1:2, :]

            m_new = jnp.maximum(m_acc, m_o)
            alpha = jnp.exp(m_acc - m_new)
            beta = jnp.exp(m_o - m_new)
            ctx_acc = alpha * ctx_acc + beta * ctx_o
            l_acc = alpha * l_acc + beta * l_o
            m_acc = m_new

        ctx_n = (ctx_acc / l_acc).astype(jnp.bfloat16)
        in_copies[4].wait()
        wo = wo_vm[...].astype(jnp.bfloat16)
        for b in range(B):
            acc = jnp.zeros((SQ, D_MODEL), jnp.float32)
            for hh in range(HQ):
                acc = acc + lax.dot_general(
                    ctx_n[b * HQ + hh], wo[hh * DH:(hh + 1) * DH, :],
                    (((0,), (0,)), ((), ())),
                    preferred_element_type=jnp.float32)
            out_ref[b, :, :] = acc

    return pl.pallas_call(
        body,
        out_shape=jax.ShapeDtypeStruct((B, SQ, D_MODEL), jnp.float32),
        in_specs=[pl.BlockSpec(memory_space=pl.ANY)] * 5,
        out_specs=pl.BlockSpec(memory_space=pltpu.VMEM),
        scratch_shapes=[
            pltpu.VMEM((B, SQ, D_MODEL), jnp.float32),
            pltpu.VMEM((D_MODEL, D_QK), jnp.float32),
            pltpu.VMEM((B, CH, HQ, DH), jnp.float32),
            pltpu.VMEM((B, CH, HQ, DH), jnp.float32),
            pltpu.VMEM((D_QK, D_MODEL), jnp.float32),
            pltpu.VMEM((2, NBH, DH, SQ), jnp.bfloat16),
            pltpu.VMEM((2, NBH, DH, SQ), jnp.bfloat16),
            pltpu.VMEM((2, NBH, 2, SQ), jnp.float32),
            pltpu.VMEM((2, NBH, 2, SQ), jnp.float32),
            pltpu.SemaphoreType.DMA((5,)),
            pltpu.SemaphoreType.DMA((2,)),
            pltpu.SemaphoreType.DMA((2,)),
            pltpu.SemaphoreType.DMA((2,)),
            pltpu.SemaphoreType.DMA((2,)),
        ],
        compiler_params=pltpu.CompilerParams(collective_id=0),
    )(x, Wq, K_ext, V_ext, Wo)
